# Initial kernel scaffold; baseline (speedup 1.0000x reference)
import jax
import jax.numpy as jnp
from jax.experimental import pallas as pl


def kernel(input, pos, token_table, pos_table):
    raise NotImplementedError("write your pallas kernel here")


# speedup vs baseline: 1.5113x; 1.5113x over previous
import functools

import jax
import jax.numpy as jnp
from jax import lax
from jax.experimental import pallas as pl
from jax.experimental.pallas import tpu as pltpu
from jax.experimental.pallas import tpu_sc as plsc

VOCAB = 100000
N_POS = 4096
D = 768
N = 4 * 4096
NC, NS = 2, 16
NW = NC * NS
PER_W = N // NW
C = 64
NCH = PER_W // C
LANES = 16
COLS = D // LANES


def _body(inp_ref, pos_ref, tok_tab, pos_tab, out_ref,
          idx_t, idx_p, tok_buf, pos_buf, sem0, sem1):
    wid = lax.axis_index("s") * NC + lax.axis_index("c")
    pltpu.sync_copy(inp_ref.at[pl.ds(wid * NCH, NCH)], idx_t)
    pltpu.sync_copy(pos_ref.at[pl.ds(wid * NCH, NCH)], idx_p)

    def chunk(j, _):
        c0 = pltpu.async_copy(tok_tab.at[idx_t.at[j]], tok_buf, sem0)
        c1 = pltpu.async_copy(pos_tab.at[idx_p.at[j]], pos_buf, sem1)
        c0.wait()
        c1.wait()

        def add_row(r, _):
            for k in range(COLS):
                s = pl.ds(k * LANES, LANES)
                tok_buf[r, s] = tok_buf[r, s] + pos_buf[r, s]
            return 0

        lax.fori_loop(0, C, add_row, 0)
        pltpu.sync_copy(tok_buf, out_ref.at[pl.ds(wid * PER_W + j * C, C)])
        return 0

    lax.fori_loop(0, NCH, chunk, 0)


@jax.jit
def kernel(input, pos, token_table, pos_table):
    mesh = plsc.VectorSubcoreMesh(core_axis_name="c", subcore_axis_name="s")
    k = pl.kernel(
        _body,
        mesh=mesh,
        out_type=jax.ShapeDtypeStruct((N, D), jnp.float32),
        scratch_types=[
            pltpu.VMEM((NCH, C), jnp.int32),
            pltpu.VMEM((NCH, C), jnp.int32),
            pltpu.VMEM((C, D), jnp.float32),
            pltpu.VMEM((C, D), jnp.float32),
            pltpu.SemaphoreType.DMA,
            pltpu.SemaphoreType.DMA,
        ],
    )
    inp2 = input.reshape(N // C, C)
    pos2 = pos.reshape(N // C, C)
    out = k(inp2, pos2, token_table, pos_table)
    return out.reshape(input.shape[0], input.shape[1], D)


# speedup vs baseline: 1.9171x; 1.2685x over previous
import jax
import jax.numpy as jnp
from jax import lax
from jax.experimental import pallas as pl
from jax.experimental.pallas import tpu as pltpu
from jax.experimental.pallas import tpu_sc as plsc

D = 768
N = 4 * 4096
NC, NS = 2, 16
NW = NC * NS
PER_W = N // NW
C = 32
NCH = PER_W // C
LANES = 16
COLS = D // LANES


def _body(inp_ref, pos_ref, tok_tab, pos_tab, out_ref,
          idx_t, idx_p, tok0, tok1, pbuf0, pbuf1,
          st0, st1, sp0, sp1):
    wid = lax.axis_index("s") * NC + lax.axis_index("c")
    pltpu.sync_copy(inp_ref.at[pl.ds(wid * NCH, NCH)], idx_t)
    pltpu.sync_copy(pos_ref.at[pl.ds(wid * NCH, NCH)], idx_p)

    toks = (tok0, tok1)
    pbufs = (pbuf0, pbuf1)
    sts = (st0, st1)
    sps = (sp0, sp1)

    def issue(j, b):
        ct = pltpu.async_copy(tok_tab.at[idx_t.at[j]], toks[b], sts[b])
        cp = pltpu.async_copy(pos_tab.at[idx_p.at[j]], pbufs[b], sps[b])
        return ct, cp

    pending = issue(0, 0)
    for j in range(NCH):
        b = j % 2
        ct, cp = pending
        ct.wait()
        cp.wait()
        if j + 1 < NCH:
            pending = issue(j + 1, 1 - b)
        tb, pb = toks[b], pbufs[b]

        def add_row(r, _, tb=tb, pb=pb):
            for k in range(COLS):
                s = pl.ds(k * LANES, LANES)
                tb[r, s] = tb[r, s] + pb[r, s]
            return 0

        lax.fori_loop(0, C, add_row, 0)
        pltpu.sync_copy(tb, out_ref.at[pl.ds(wid * PER_W + j * C, C)])


@jax.jit
def kernel(input, pos, token_table, pos_table):
    mesh = plsc.VectorSubcoreMesh(core_axis_name="c", subcore_axis_name="s")
    k = pl.kernel(
        _body,
        mesh=mesh,
        out_type=jax.ShapeDtypeStruct((N, D), jnp.float32),
        scratch_types=[
            pltpu.VMEM((NCH, C), jnp.int32),
            pltpu.VMEM((NCH, C), jnp.int32),
            pltpu.VMEM((C, D), jnp.float32),
            pltpu.VMEM((C, D), jnp.float32),
            pltpu.VMEM((C, D), jnp.float32),
            pltpu.VMEM((C, D), jnp.float32),
            pltpu.SemaphoreType.DMA,
            pltpu.SemaphoreType.DMA,
            pltpu.SemaphoreType.DMA,
            pltpu.SemaphoreType.DMA,
        ],
    )
    inp2 = input.reshape(N // C, C)
    pos2 = pos.reshape(N // C, C)
    out = k(inp2, pos2, token_table, pos_table)
    return out.reshape(input.shape[0], input.shape[1], D)
